# Initial kernel scaffold; baseline (speedup 1.0000x reference)
import jax
import jax.numpy as jnp
from jax.experimental import pallas as pl


def kernel(x, edge_index, W1, b1, W2, b2):
    raise NotImplementedError("write your pallas kernel here")


# speedup vs baseline: 3.7558x; 3.7558x over previous
import functools

import jax
import jax.numpy as jnp
from jax import lax
from jax.experimental import pallas as pl
from jax.experimental.pallas import tpu as pltpu
from jax.experimental.pallas import tpu_sc as plsc

N = 10000
E = 320000
D = 128

NC = 2
NS = 16
NW = NC * NS
CHUNK = 128
ITERS = 79
NCHUNKS = NW * ITERS
EPAD = NCHUNKS * CHUNK
NPAD = 10240
ROWS_PER_SUB = NPAD // NS
SWEEP = ROWS_PER_SUB // CHUNK
DUMMY_ROW = NPAD - 1

_MESH = dict(core_axis_name="c", subcore_axis_name="s",
             num_cores=NC, num_subcores=NS)


def _ids():
  cid = lax.axis_index("c")
  sid = lax.axis_index("s")
  return cid, sid, sid * NC + cid, sid * ROWS_PER_SUB


def _zero_acc(zrows_hbm, rows_v, acc_sh, row0):
  pltpu.sync_copy(zrows_hbm, rows_v)
  for k in range(SWEEP):
    pltpu.sync_copy(rows_v, acc_sh.at[pl.ds(row0 + k * CHUNK, CHUNK)])


def _write_back(acc_sh, rows_v, out_hbm, cid, row0):
  for k in range(SWEEP):
    r = row0 + k * CHUNK
    pltpu.sync_copy(acc_sh.at[pl.ds(r, CHUNK)], rows_v)
    pltpu.sync_copy(rows_v, out_hbm.at[cid, pl.ds(r, CHUNK)])


@functools.lru_cache(maxsize=None)
def _make_sc_agg():
  out_type = [jax.ShapeDtypeStruct((NC, NPAD, D), jnp.float32)]
  scratch = [
      pltpu.VMEM((CHUNK,), jnp.int32),
      pltpu.VMEM((CHUNK,), jnp.int32),
      pltpu.VMEM((CHUNK, D), jnp.float32),
      pltpu.VMEM_SHARED((NPAD, D), jnp.float32),
  ]

  def body(h_hbm, src_hbm, dst_hbm, zrows_hbm, agg_hbm,
           sidx_v, didx_v, rows_v, acc_sh):
    cid, sid, wid, row0 = _ids()
    _zero_acc(zrows_hbm, rows_v, acc_sh, row0)
    plsc.subcore_barrier()

    def step(j, carry):
      c = wid + j * NW
      pltpu.sync_copy(src_hbm.at[c], sidx_v)
      pltpu.sync_copy(dst_hbm.at[c], didx_v)
      pltpu.sync_copy(h_hbm.at[sidx_v], rows_v)
      pltpu.sync_copy(rows_v, acc_sh.at[didx_v], add=True)
      return carry

    lax.fori_loop(0, ITERS, step, 0)
    plsc.subcore_barrier()
    _write_back(acc_sh, rows_v, agg_hbm, cid, row0)

  return pl.kernel(body, out_type=out_type,
                   mesh=plsc.VectorSubcoreMesh(**_MESH),
                   scratch_types=scratch)


@functools.lru_cache(maxsize=None)
def _make_sc_deg():
  out_type = [jax.ShapeDtypeStruct((NC, NPAD, D), jnp.float32)]
  scratch = [
      pltpu.VMEM((CHUNK,), jnp.int32),
      pltpu.VMEM((CHUNK, D), jnp.float32),
      pltpu.VMEM_SHARED((NPAD, D), jnp.float32),
  ]

  def body(dst_hbm, zrows_hbm, ones_hbm, deg_hbm, didx_v, rows_v, acc_sh):
    cid, sid, wid, row0 = _ids()
    _zero_acc(zrows_hbm, rows_v, acc_sh, row0)
    plsc.subcore_barrier()
    pltpu.sync_copy(ones_hbm, rows_v)

    def step(j, carry):
      c = wid + j * NW
      pltpu.sync_copy(dst_hbm.at[c], didx_v)
      pltpu.sync_copy(rows_v, acc_sh.at[didx_v], add=True)
      return carry

    lax.fori_loop(0, ITERS, step, 0)
    plsc.subcore_barrier()
    _write_back(acc_sh, rows_v, deg_hbm, cid, row0)

  return pl.kernel(body, out_type=out_type,
                   mesh=plsc.VectorSubcoreMesh(**_MESH),
                   scratch_types=scratch)


_TC_R = 512


def _tc_layer_body(h_ref, p0_ref, p1_ref, d0_ref, d1_ref, w_ref,
                   b_ref, o_ref):
  inv = 1.0 / (d0_ref[:, 0:1] + d1_ref[:, 0:1] + 1.0)
  s = (h_ref[...] + p0_ref[...] + p1_ref[...]) * inv
  o = jnp.dot(s, w_ref[...], preferred_element_type=jnp.float32) + b_ref[...]
  o_ref[...] = jnp.maximum(o, 0.0)


def _tc_layer(h, p0, p1, d0, d1, W, b2d):
  return pl.pallas_call(
      _tc_layer_body,
      grid=(NPAD // _TC_R,),
      in_specs=[
          pl.BlockSpec((_TC_R, D), lambda i: (i, 0)),
          pl.BlockSpec((_TC_R, D), lambda i: (i, 0)),
          pl.BlockSpec((_TC_R, D), lambda i: (i, 0)),
          pl.BlockSpec((_TC_R, D), lambda i: (i, 0)),
          pl.BlockSpec((_TC_R, D), lambda i: (i, 0)),
          pl.BlockSpec((D, D), lambda i: (0, 0)),
          pl.BlockSpec((1, D), lambda i: (0, 0)),
      ],
      out_specs=pl.BlockSpec((_TC_R, D), lambda i: (i, 0)),
      out_shape=jax.ShapeDtypeStruct((N, D), jnp.float32),
  )(h, p0, p1, d0, d1, W, b2d)


def kernel(x, edge_index, W1, b1, W2, b2):
  npad = EPAD - E
  src = jnp.concatenate(
      [edge_index[0].astype(jnp.int32), jnp.zeros((npad,), jnp.int32)])
  dst = jnp.concatenate(
      [edge_index[1].astype(jnp.int32),
       jnp.full((npad,), DUMMY_ROW, jnp.int32)])
  src = src.reshape(NCHUNKS, CHUNK)
  dst = dst.reshape(NCHUNKS, CHUNK)
  zrows = jnp.zeros((CHUNK, D), jnp.float32)
  ones = jnp.ones((CHUNK, D), jnp.float32)

  (deg,) = _make_sc_deg()(dst, zrows, ones)
  sc_agg = _make_sc_agg()
  (agg1,) = sc_agg(x, src, dst, zrows)
  h1 = _tc_layer(x, agg1[0], agg1[1], deg[0], deg[1], W1, b1.reshape(1, D))
  (agg2,) = sc_agg(h1, src, dst, zrows)
  h2 = _tc_layer(h1, agg2[0], agg2[1], deg[0], deg[1], W2, b2.reshape(1, D))
  return h2
